# baseline (speedup 1.0000x reference)
import functools

import jax
import jax.numpy as jnp
from jax import lax
from jax.experimental import pallas as pl
from jax.experimental.pallas import tpu as pltpu
from jax.experimental.pallas import tpu_sc as plsc

F = 100
M = 20
H1 = 64
H2 = 32
PAIR_LANES = 2 * M
GRP = 4
NG = M // GRP
H1F = M * H1
H2F = M * H2

NW = 16
RS = 128
WPAD = 104 * RS
CHUNK = WPAD // NW
NEGF = -3.0e38
BIGI = 2**30


def _sc_topk(w_hbm, vals_hbm, si_hbm, sj_hbm,
             w_buf, lvals, lidx, mv, mi, outv, shv, shi):
    wid = lax.axis_index("s")
    base = wid * CHUNK
    lane = lax.iota(jnp.int32, 16)

    pltpu.sync_copy(w_hbm.at[pl.ds(base, CHUNK)], w_buf)

    def _perm(x, idx):
        return lax.gather(
            x, idx[:, None],
            lax.GatherDimensionNumbers(
                offset_dims=(), collapsed_slice_dims=(0,),
                start_index_map=(0,)),
            (1,), mode=lax.GatherScatterMode.PROMISE_IN_BOUNDS)

    def _bmax(x):
        for s in (1, 2, 4, 8):
            x = jnp.maximum(x, _perm(x, lane ^ s))
        return x

    def _bmin(x):
        for s in (1, 2, 4, 8):
            x = jnp.minimum(x, _perm(x, lane ^ s))
        return x

    def _top_m(nvec, loadw, loadi):
        acc = [jnp.full((16,), NEGF, jnp.float32),
               jnp.full((16,), NEGF, jnp.float32)]
        aci = [jnp.full((16,), BIGI, jnp.int32),
               jnp.full((16,), BIGI, jnp.int32)]
        vk = jnp.full((16,), 3.0e38, jnp.float32)
        ik = jnp.full((16,), -1, jnp.int32)
        for k in range(M):
            def _scan(j, carry):
                rv, ri, vkc, ikc = carry
                w = loadw(j)
                li = loadi(j)
                elig = (w < vkc) | ((w == vkc) & (li > ikc))
                w = jnp.where(elig, w, NEGF)
                li = jnp.where(elig, li, BIGI)
                better = (w > rv) | ((w == rv) & (li < ri))
                return (jnp.where(better, w, rv),
                        jnp.where(better, li, ri), vkc, ikc)

            rv, ri, _, _ = lax.fori_loop(
                0, nvec, _scan,
                (jnp.full((16,), NEGF, jnp.float32),
                 jnp.full((16,), BIGI, jnp.int32), vk, ik))
            bv = _bmax(rv)
            bi = _bmin(jnp.where(rv == bv, ri, BIGI))
            half, ln = divmod(k, 16)
            acc[half] = jnp.where(lane == ln, bv, acc[half])
            aci[half] = jnp.where(lane == ln, bi, aci[half])
            vk = bv
            ik = bi
        return acc, aci

    acc, aci = _top_m(
        CHUNK // 16,
        lambda j: w_buf[pl.ds(j * 16, 16)],
        lambda j: base + j * 16 + lane)

    lvals[pl.ds(0, 16)] = acc[0]
    lvals[pl.ds(16, 16)] = acc[1]
    lidx[pl.ds(0, 16)] = aci[0]
    lidx[pl.ds(16, 16)] = aci[1]

    pltpu.sync_copy(lvals, shv.at[pl.ds(wid * 32, 32)])
    pltpu.sync_copy(lidx, shi.at[pl.ds(wid * 32, 32)])
    plsc.subcore_barrier()

    @pl.when(wid == 0)
    def _merge():
        pltpu.sync_copy(shv, mv)
        pltpu.sync_copy(shi, mi)
        ov, oi = _top_m(
            (NW * 32) // 16,
            lambda j: mv[pl.ds(j * 16, 16)],
            lambda j: mi[pl.ds(j * 16, 16)])

        outv[pl.ds(0, 16)] = 1.0 / (1.0 + jnp.exp(-ov[0]))
        outv[pl.ds(16, 16)] = 1.0 / (1.0 + jnp.exp(-ov[1]))
        mi[pl.ds(0, 16)] = oi[0] >> 7
        mi[pl.ds(16, 16)] = oi[1] >> 7
        lidx[pl.ds(0, 16)] = oi[0] & (RS - 1)
        lidx[pl.ds(16, 16)] = oi[1] & (RS - 1)

        pltpu.sync_copy(outv, vals_hbm)
        pltpu.sync_copy(mi.at[pl.ds(0, 32)], si_hbm)
        pltpu.sync_copy(lidx, sj_hbm)


_sc_mesh = plsc.VectorSubcoreMesh(
    core_axis_name="c", subcore_axis_name="s", num_cores=1)

_sc_topk_call = functools.partial(
    pl.kernel,
    mesh=_sc_mesh,
    out_type=[
        jax.ShapeDtypeStruct((32,), jnp.float32),
        jax.ShapeDtypeStruct((32,), jnp.int32),
        jax.ShapeDtypeStruct((32,), jnp.int32),
    ],
    scratch_types=[
        pltpu.VMEM((CHUNK,), jnp.float32),
        pltpu.VMEM((32,), jnp.float32),
        pltpu.VMEM((32,), jnp.int32),
        pltpu.VMEM((NW * 32,), jnp.float32),
        pltpu.VMEM((NW * 32,), jnp.int32),
        pltpu.VMEM((32,), jnp.float32),
        pltpu.VMEM_SHARED((NW * 32,), jnp.float32),
        pltpu.VMEM_SHARED((NW * 32,), jnp.int32),
    ],
)(_sc_topk)


def _main_body(x_ref, sel_ref, W1row_ref, b1f_ref, W2bd_ref, b2f_ref,
               W3col_ref, b3_ref, Wc1T_ref, bc1_ref, Wc2T_ref, bc2_ref,
               feat_ref, csum_ref, G_ref, *, num_tiles, inv_b):
    pid = pl.program_id(0)

    @pl.when(pid == 0)
    def _build_g():
        frow = jax.lax.broadcasted_iota(jnp.int32, (F, PAIR_LANES), 0)
        S = (frow == jnp.broadcast_to(sel_ref[...], (F, PAIR_LANES))).astype(
            jnp.float32)
        G_ref[...] = jnp.dot(S, W1row_ref[...],
                             preferred_element_type=jnp.float32)

    xt = x_ref[...]
    hc = jnp.maximum(
        jnp.dot(xt, Wc1T_ref[...], preferred_element_type=jnp.float32)
        + bc1_ref[...], 0.0)
    cw = jax.nn.sigmoid(
        jnp.dot(hc, Wc2T_ref[...], preferred_element_type=jnp.float32)
        + bc2_ref[...])

    h1 = jnp.maximum(
        jnp.dot(xt, G_ref[...], preferred_element_type=jnp.float32)
        + b1f_ref[...], 0.0)
    h2g = []
    for g in range(NG):
        hg = jnp.dot(h1[:, g * GRP * H1:(g + 1) * GRP * H1], W2bd_ref[g],
                     preferred_element_type=jnp.float32)
        h2g.append(jnp.maximum(
            hg + b2f_ref[:, g * GRP * H2:(g + 1) * GRP * H2], 0.0))
    h2 = jnp.concatenate(h2g, axis=1)
    o = jnp.dot(h2, W3col_ref[...], preferred_element_type=jnp.float32)
    feat_ref[...] = (o + b3_ref[...]) * cw

    @pl.when(pid == 0)
    def _init():
        csum_ref[...] = jnp.zeros_like(csum_ref)

    csum_ref[...] += jnp.sum(cw, axis=0, keepdims=True)

    @pl.when(pid == num_tiles - 1)
    def _fin():
        csum_ref[...] *= inv_b


@jax.jit
def kernel(x, W_int, W1, b1, W2, b2, W3, b3, Wc1, bc1, Wc2, bc2):
    B = x.shape[0]
    T = 2048
    n = B // T
    eyeM = jnp.eye(M, dtype=jnp.float32)
    W1a = (W1[:, None, :, 0] * eyeM[:, :, None]).reshape(M, H1F)
    W1b = (W1[:, None, :, 1] * eyeM[:, :, None]).reshape(M, H1F)
    W1row = jnp.concatenate([W1a, W1b], axis=0)
    b1f = b1.reshape(1, H1F)
    W2T = jnp.transpose(W2, (0, 2, 1)).reshape(NG, GRP, H1, H2)
    eyeG = jnp.eye(GRP, dtype=jnp.float32)
    W2bd = (W2T[:, :, :, None, :] *
            eyeG[None, :, None, :, None]).reshape(NG, GRP * H1, GRP * H2)
    b2f = b2.reshape(1, H2F)
    W3col = (W3[:, 0, :, None] * eyeM[:, None, :]).reshape(H2F, M)
    b3r = jnp.reshape(b3, (1, M))
    Wc1T = Wc1.T
    Wc2T = Wc2.T
    bc1r = bc1.reshape(1, H1)
    bc2r = bc2.reshape(1, M)

    row = jax.lax.broadcasted_iota(jnp.int32, (F, F), 0)
    col = jax.lax.broadcasted_iota(jnp.int32, (F, F), 1)
    wmask = jnp.where(col > row, W_int, NEGF)
    wflat = jnp.pad(wmask, ((0, 104 - F), (0, RS - F)),
                    constant_values=NEGF).reshape(WPAD)
    vals, si, sj = _sc_topk_call(wflat)
    sel = jnp.concatenate([si[:M], sj[:M]]).reshape(1, PAIR_LANES)

    feat, cmean = pl.pallas_call(
        functools.partial(_main_body, num_tiles=n, inv_b=1.0 / B),
        grid=(n,),
        in_specs=[
            pl.BlockSpec((T, F), lambda i: (i, 0)),
            pl.BlockSpec((1, PAIR_LANES), lambda i: (0, 0)),
            pl.BlockSpec((PAIR_LANES, H1F), lambda i: (0, 0)),
            pl.BlockSpec((1, H1F), lambda i: (0, 0)),
            pl.BlockSpec((NG, GRP * H1, GRP * H2), lambda i: (0, 0, 0)),
            pl.BlockSpec((1, H2F), lambda i: (0, 0)),
            pl.BlockSpec((H2F, M), lambda i: (0, 0)),
            pl.BlockSpec((1, M), lambda i: (0, 0)),
            pl.BlockSpec((F, H1), lambda i: (0, 0)),
            pl.BlockSpec((1, H1), lambda i: (0, 0)),
            pl.BlockSpec((H1, M), lambda i: (0, 0)),
            pl.BlockSpec((1, M), lambda i: (0, 0)),
        ],
        out_specs=[
            pl.BlockSpec((T, M), lambda i: (i, 0)),
            pl.BlockSpec((1, M), lambda i: (0, 0)),
        ],
        out_shape=[
            jax.ShapeDtypeStruct((B, M), jnp.float32),
            jax.ShapeDtypeStruct((1, M), jnp.float32),
        ],
        scratch_shapes=[pltpu.VMEM((F, H1F), jnp.float32)],
    )(x, sel, W1row, b1f, W2bd, b2f, W3col, b3r, Wc1T, bc1r, Wc2T, bc2r)
    selected_pairs = jnp.stack([si[:M], sj[:M]], axis=1)
    return (feat, vals[:M], cmean[0], selected_pairs)
